# Initial kernel scaffold; baseline (speedup 1.0000x reference)
import jax
import jax.numpy as jnp
from jax.experimental import pallas as pl


def kernel(indices, table):
    raise NotImplementedError("write your pallas kernel here")


# speedup vs baseline: 1.0418x; 1.0418x over previous
import functools

import jax
import jax.numpy as jnp
from jax import lax
from jax.experimental import pallas as pl
from jax.experimental.pallas import tpu as pltpu
from jax.experimental.pallas import tpu_sc as plsc

VOCAB = 1000002
EMBED_DIM = 32
N_TOKENS = 819200

_info = plsc.get_sparse_core_info()
_NW = _info.num_cores * _info.num_subcores
_B_PER_W = N_TOKENS // _NW
_CHUNK = 1024
_N_CHUNKS = _B_PER_W // _CHUNK


def _embed_body(idx_hbm, table_hbm, out_hbm, idx_v, rows_v, sem):
    wid = lax.axis_index("s") * _info.num_cores + lax.axis_index("c")
    base = wid * _B_PER_W

    def step(c, carry):
        off = base + c * _CHUNK
        pltpu.sync_copy(idx_hbm.at[pl.ds(off, _CHUNK)], idx_v)
        pltpu.async_copy(table_hbm.at[idx_v], rows_v, sem).wait()
        pltpu.sync_copy(rows_v, out_hbm.at[pl.ds(off, _CHUNK)])
        return carry

    lax.fori_loop(0, _N_CHUNKS, step, 0)


@jax.jit
def _embed(indices, table):
    mesh = plsc.VectorSubcoreMesh(core_axis_name="c", subcore_axis_name="s")
    f = functools.partial(
        pl.kernel,
        mesh=mesh,
        out_type=jax.ShapeDtypeStruct((N_TOKENS, EMBED_DIM), jnp.float32),
        scratch_types=[
            pltpu.VMEM((_CHUNK,), jnp.int32),
            pltpu.VMEM((_CHUNK, EMBED_DIM), jnp.float32),
            pltpu.SemaphoreType.DMA,
        ],
        compiler_params=pltpu.CompilerParams(use_tc_tiling_on_sc=False),
    )(_embed_body)
    return f(indices, table)


def kernel(indices, table):
    return _embed(indices, table)


# speedup vs baseline: 1.0717x; 1.0288x over previous
import functools

import jax
import jax.numpy as jnp
from jax import lax
from jax.experimental import pallas as pl
from jax.experimental.pallas import tpu as pltpu
from jax.experimental.pallas import tpu_sc as plsc

VOCAB = 1000002
EMBED_DIM = 32
N_TOKENS = 819200

_info = plsc.get_sparse_core_info()
_NW = _info.num_cores * _info.num_subcores
_B_PER_W = N_TOKENS // _NW
_CHUNK = 1600
_N_CHUNKS = _B_PER_W // _CHUNK


def _embed_body(idx_hbm, table_hbm, out_hbm, idx_v, rows_v, gsem0, gsem1,
                ssem0, ssem1):
    wid = lax.axis_index("s") * _info.num_cores + lax.axis_index("c")
    base = wid * _B_PER_W
    gsems = (gsem0, gsem1)
    ssems = (ssem0, ssem1)

    def start_gather(g, b):
        idx_ref = idx_v.at[pl.ds(g * _CHUNK, _CHUNK)]
        pltpu.make_async_copy(table_hbm.at[idx_ref], rows_v.at[b],
                              gsems[b]).start()

    def wait_gather(b):
        pltpu.make_async_copy(out_hbm.at[pl.ds(0, _CHUNK)], rows_v.at[b],
                              gsems[b]).wait()

    def start_store(g, b):
        pltpu.make_async_copy(rows_v.at[b],
                              out_hbm.at[pl.ds(base + g * _CHUNK, _CHUNK)],
                              ssems[b]).start()

    def wait_store(b):
        pltpu.make_async_copy(out_hbm.at[pl.ds(0, _CHUNK)], rows_v.at[b],
                              ssems[b]).wait()

    pltpu.sync_copy(idx_hbm.at[pl.ds(base, _B_PER_W)], idx_v)

    start_gather(0, 0)
    start_gather(1, 1)
    wait_gather(0)
    start_store(0, 0)

    def superstep(s, carry):
        for p in (1, 2):
            g = 2 * s + p
            b = p & 1
            b2 = 1 - b
            wait_store(b2)
            start_gather(g + 1, b2)
            wait_gather(b)
            start_store(g, b)
        return carry

    lax.fori_loop(0, (_N_CHUNKS - 2) // 2, superstep, 0)

    bl = (_N_CHUNKS - 1) & 1
    wait_gather(bl)
    start_store(_N_CHUNKS - 1, bl)
    wait_store(1 - bl)
    wait_store(bl)


@jax.jit
def _embed(indices, table):
    mesh = plsc.VectorSubcoreMesh(core_axis_name="c", subcore_axis_name="s")
    f = functools.partial(
        pl.kernel,
        mesh=mesh,
        out_type=jax.ShapeDtypeStruct((N_TOKENS, EMBED_DIM), jnp.float32),
        scratch_types=[
            pltpu.VMEM((_B_PER_W,), jnp.int32),
            pltpu.VMEM((2, _CHUNK, EMBED_DIM), jnp.float32),
            pltpu.SemaphoreType.DMA,
            pltpu.SemaphoreType.DMA,
            pltpu.SemaphoreType.DMA,
            pltpu.SemaphoreType.DMA,
        ],
        compiler_params=pltpu.CompilerParams(use_tc_tiling_on_sc=False),
    )(_embed_body)
    return f(indices, table)


def kernel(indices, table):
    return _embed(indices, table)
